# baseline (speedup 1.0000x reference)
import functools

import jax
import jax.numpy as jnp
from jax import lax
from jax.experimental import pallas as pl
from jax.experimental.pallas import tpu as pltpu
from jax.experimental.pallas import tpu_sc as plsc

TOP_K = 500
CROP = 14
C = 256
EPS = 1e-7

LEVEL_H = (64, 32, 16, 8, 4)
LEVEL_BASE = (0, 4096, 5120, 5376, 5440)

NC = 2
NS = 16
NW = NC * NS

S = TOP_K * CROP * CROP
S_PER_W = 3072
CHUNK = 64
N_CHUNKS = S_PER_W // CHUNK


def _roi_body(table, i00, i01, i10, i11, w00, w01, w10, w11, out,
              ib00, ib01, ib10, ib11, wb00, wb01, wb10, wb11,
              v00, v01, v10, v11, ob, sem):
    wid = lax.axis_index("s") * NC + lax.axis_index("c")
    start = jnp.minimum(wid * S_PER_W, S - S_PER_W)

    def chunk_body(ci, carry):
        off = start + ci * CHUNK
        sl = pl.ds(off, CHUNK)
        pltpu.sync_copy(i00.at[sl], ib00)
        pltpu.sync_copy(i01.at[sl], ib01)
        pltpu.sync_copy(i10.at[sl], ib10)
        pltpu.sync_copy(i11.at[sl], ib11)
        pltpu.sync_copy(w00.at[sl], wb00.at[pl.ds(0, CHUNK)])
        pltpu.sync_copy(w01.at[sl], wb01.at[pl.ds(0, CHUNK)])
        pltpu.sync_copy(w10.at[sl], wb10.at[pl.ds(0, CHUNK)])
        pltpu.sync_copy(w11.at[sl], wb11.at[pl.ds(0, CHUNK)])
        d0 = pltpu.async_copy(table.at[ib00], v00, sem)
        d1 = pltpu.async_copy(table.at[ib01], v01, sem)
        d2 = pltpu.async_copy(table.at[ib10], v10, sem)
        d3 = pltpu.async_copy(table.at[ib11], v11, sem)
        d0.wait()
        d1.wait()
        d2.wait()
        d3.wait()

        @plsc.parallel_loop(0, CHUNK, 1, unroll=2)
        def sample_body(s):
            a = jnp.full((16,), wb00[pl.ds(s, 16)][0], dtype=jnp.float32)
            b = jnp.full((16,), wb01[pl.ds(s, 16)][0], dtype=jnp.float32)
            c = jnp.full((16,), wb10[pl.ds(s, 16)][0], dtype=jnp.float32)
            d = jnp.full((16,), wb11[pl.ds(s, 16)][0], dtype=jnp.float32)
            for cc in range(C // 16):
                csl = pl.ds(cc * 16, 16)
                acc = a * v00[s, csl] + b * v01[s, csl]
                acc = acc + c * v10[s, csl] + d * v11[s, csl]
                ob[s, csl] = acc

        pltpu.sync_copy(ob, out.at[sl])
        return carry

    lax.fori_loop(0, N_CHUNKS, chunk_body, 0, unroll=False)


@jax.jit
def _roi_gather(table, i00, i01, i10, i11, w00, w01, w10, w11):
    mesh = plsc.VectorSubcoreMesh(core_axis_name="c", subcore_axis_name="s",
                                  num_cores=NC)
    return pl.kernel(
        _roi_body,
        out_type=jax.ShapeDtypeStruct((S, C), jnp.float32),
        mesh=mesh,
        scratch_types=[
            pltpu.VMEM((CHUNK,), jnp.int32),
            pltpu.VMEM((CHUNK,), jnp.int32),
            pltpu.VMEM((CHUNK,), jnp.int32),
            pltpu.VMEM((CHUNK,), jnp.int32),
            pltpu.VMEM((CHUNK + 16,), jnp.float32),
            pltpu.VMEM((CHUNK + 16,), jnp.float32),
            pltpu.VMEM((CHUNK + 16,), jnp.float32),
            pltpu.VMEM((CHUNK + 16,), jnp.float32),
            pltpu.VMEM((CHUNK, C), jnp.float32),
            pltpu.VMEM((CHUNK, C), jnp.float32),
            pltpu.VMEM((CHUNK, C), jnp.float32),
            pltpu.VMEM((CHUNK, C), jnp.float32),
            pltpu.VMEM((CHUNK, C), jnp.float32),
            pltpu.SemaphoreType.DMA,
        ],
    )(table, i00, i01, i10, i11, w00, w01, w10, w11)


_RB = 4


def _retile_body(flat_ref, out_ref):
    out_ref[...] = flat_ref[...].reshape(1, _RB, CROP, CROP, C)


@jax.jit
def _retile(flat):
    return pl.pallas_call(
        _retile_body,
        grid=(TOP_K // _RB,),
        in_specs=[pl.BlockSpec((_RB * CROP * CROP, C), lambda b: (b, 0))],
        out_specs=pl.BlockSpec((1, _RB, CROP, CROP, C),
                               lambda b: (0, b, 0, 0, 0)),
        out_shape=jax.ShapeDtypeStruct((1, TOP_K, CROP, CROP, C), jnp.float32),
    )(flat)


def _map_to_level(boxes):
    w = boxes[:, 2] - boxes[:, 0]
    h = boxes[:, 3] - boxes[:, 1]
    size = jnp.sqrt(w * h)
    levels = jnp.floor(1.0 + jnp.log2(size / 224.0 + EPS))
    return jnp.clip(levels, 0.0, 4.0)


def kernel(image_shape, boxes, classification, p0, p1, p2, p3, p4):
    table = jnp.concatenate(
        [p.reshape(-1, C) for p in (p0[0], p1[0], p2[0], p3[0], p4[0])], axis=0)

    b = boxes[0]
    cls = classification[0]
    scores = jnp.max(cls, axis=1)
    _, idx = lax.top_k(scores, TOP_K)
    b = jnp.take(b, idx, axis=0)
    cls = jnp.take(cls, idx, axis=0)
    levels = _map_to_level(b)
    order = jnp.argsort(levels, stable=True)
    b = jnp.take(b, order, axis=0)
    cls = jnp.take(cls, order, axis=0)
    levels = jnp.take(levels, order, axis=0)

    Hf = image_shape[1].astype(jnp.float32)
    Wf = image_shape[2].astype(jnp.float32)
    y1 = b[:, 1] / Hf
    x1 = b[:, 0] / Wf
    y2 = b[:, 3] / Hf
    x2 = b[:, 2] / Wf

    lev_i = levels.astype(jnp.int32)
    Hl = jnp.take(jnp.array(LEVEL_H, jnp.float32), lev_i)
    lbase = jnp.take(jnp.array(LEVEL_BASE, jnp.int32), lev_i)
    Wl_i = jnp.take(jnp.array(LEVEL_H, jnp.int32), lev_i)

    iy = jnp.arange(CROP, dtype=jnp.float32) / float(CROP - 1)
    ys = y1[:, None] * (Hl[:, None] - 1.0) + (y2 - y1)[:, None] * (Hl[:, None] - 1.0) * iy[None, :]
    xs = x1[:, None] * (Hl[:, None] - 1.0) + (x2 - x1)[:, None] * (Hl[:, None] - 1.0) * iy[None, :]
    y0f = jnp.floor(ys)
    x0f = jnp.floor(xs)
    Hi = Wl_i[:, None]
    y0 = jnp.clip(y0f.astype(jnp.int32), 0, Hi - 1)
    y1i = jnp.clip(y0 + 1, 0, Hi - 1)
    x0 = jnp.clip(x0f.astype(jnp.int32), 0, Hi - 1)
    x1i = jnp.clip(x0 + 1, 0, Hi - 1)
    wy = ys - y0f
    wx = xs - x0f

    row0 = lbase[:, None] + y0 * Wl_i[:, None]
    row1 = lbase[:, None] + y1i * Wl_i[:, None]
    i00 = row0[:, :, None] + x0[:, None, :]
    i01 = row0[:, :, None] + x1i[:, None, :]
    i10 = row1[:, :, None] + x0[:, None, :]
    i11 = row1[:, :, None] + x1i[:, None, :]
    wyc = wy[:, :, None]
    wxc = wx[:, None, :]
    w00 = (1.0 - wyc) * (1.0 - wxc)
    w01 = (1.0 - wyc) * wxc
    w10 = wyc * (1.0 - wxc)
    w11 = wyc * wxc

    def flat_i(a):
        return a.reshape(S).astype(jnp.int32)

    def flat_w(a):
        return jnp.broadcast_to(a, (TOP_K, CROP, CROP)).reshape(S)

    out = _roi_gather(table,
                      flat_i(i00), flat_i(i01), flat_i(i10), flat_i(i11),
                      flat_w(w00), flat_w(w01), flat_w(w10), flat_w(w11))
    rois = out.reshape(1, TOP_K, CROP, CROP, C)
    return (b[None], cls[None], rois)


# speedup vs baseline: 1.5044x; 1.5044x over previous
import functools

import jax
import jax.numpy as jnp
from jax import lax
from jax.experimental import pallas as pl
from jax.experimental.pallas import tpu as pltpu
from jax.experimental.pallas import tpu_sc as plsc

TOP_K = 500
CROP = 14
C = 256
EPS = 1e-7

LEVEL_H = (64, 32, 16, 8, 4)
LEVEL_BASE = (0, 4096, 5120, 5376, 5440)

NC = 2
NS = 16
NW = NC * NS

S = TOP_K * CROP * CROP
S_PER_W = 3072
CHUNK = 64
N_CHUNKS = S_PER_W // CHUNK


def _roi_body(table, i00, i01, i10, i11, w00, w01, w10, w11, out,
              ib00, ib01, ib10, ib11, wb00, wb01, wb10, wb11,
              v00, v01, v10, v11, ob, sem):
    wid = lax.axis_index("s") * NC + lax.axis_index("c")
    start = jnp.minimum(wid * S_PER_W, S - S_PER_W)

    def chunk_body(ci, carry):
        off = start + ci * CHUNK
        sl = pl.ds(off, CHUNK)
        pltpu.sync_copy(i00.at[sl], ib00)
        pltpu.sync_copy(i01.at[sl], ib01)
        pltpu.sync_copy(i10.at[sl], ib10)
        pltpu.sync_copy(i11.at[sl], ib11)
        pltpu.sync_copy(w00.at[sl], wb00.at[pl.ds(0, CHUNK)])
        pltpu.sync_copy(w01.at[sl], wb01.at[pl.ds(0, CHUNK)])
        pltpu.sync_copy(w10.at[sl], wb10.at[pl.ds(0, CHUNK)])
        pltpu.sync_copy(w11.at[sl], wb11.at[pl.ds(0, CHUNK)])
        d0 = pltpu.async_copy(table.at[ib00], v00, sem)
        d1 = pltpu.async_copy(table.at[ib01], v01, sem)
        d2 = pltpu.async_copy(table.at[ib10], v10, sem)
        d3 = pltpu.async_copy(table.at[ib11], v11, sem)
        d0.wait()
        d1.wait()
        d2.wait()
        d3.wait()

        @plsc.parallel_loop(0, CHUNK, 1, unroll=2)
        def sample_body(s):
            a = jnp.full((16,), wb00[pl.ds(s, 16)][0], dtype=jnp.float32)
            b = jnp.full((16,), wb01[pl.ds(s, 16)][0], dtype=jnp.float32)
            c = jnp.full((16,), wb10[pl.ds(s, 16)][0], dtype=jnp.float32)
            d = jnp.full((16,), wb11[pl.ds(s, 16)][0], dtype=jnp.float32)
            for cc in range(C // 16):
                csl = pl.ds(cc * 16, 16)
                acc = a * v00[s, csl] + b * v01[s, csl]
                acc = acc + c * v10[s, csl] + d * v11[s, csl]
                ob[pl.ds(s * C + cc * 16, 16)] = acc

        pltpu.sync_copy(ob, out.at[pl.ds(off * C, CHUNK * C)])
        return carry

    lax.fori_loop(0, N_CHUNKS, chunk_body, 0, unroll=False)


@jax.jit
def _roi_gather(table, i00, i01, i10, i11, w00, w01, w10, w11):
    mesh = plsc.VectorSubcoreMesh(core_axis_name="c", subcore_axis_name="s",
                                  num_cores=NC)
    return pl.kernel(
        _roi_body,
        out_type=jax.ShapeDtypeStruct((S * C,), jnp.float32),
        mesh=mesh,
        scratch_types=[
            pltpu.VMEM((CHUNK,), jnp.int32),
            pltpu.VMEM((CHUNK,), jnp.int32),
            pltpu.VMEM((CHUNK,), jnp.int32),
            pltpu.VMEM((CHUNK,), jnp.int32),
            pltpu.VMEM((CHUNK + 16,), jnp.float32),
            pltpu.VMEM((CHUNK + 16,), jnp.float32),
            pltpu.VMEM((CHUNK + 16,), jnp.float32),
            pltpu.VMEM((CHUNK + 16,), jnp.float32),
            pltpu.VMEM((CHUNK, C), jnp.float32),
            pltpu.VMEM((CHUNK, C), jnp.float32),
            pltpu.VMEM((CHUNK, C), jnp.float32),
            pltpu.VMEM((CHUNK, C), jnp.float32),
            pltpu.VMEM((CHUNK * C,), jnp.float32),
            pltpu.SemaphoreType.DMA,
        ],
    )(table, i00, i01, i10, i11, w00, w01, w10, w11)


_RB = 4


def _retile_body(flat_ref, out_ref):
    out_ref[...] = flat_ref[...].reshape(1, _RB, CROP, CROP, C)


@jax.jit
def _retile(flat):
    return pl.pallas_call(
        _retile_body,
        grid=(TOP_K // _RB,),
        in_specs=[pl.BlockSpec((_RB * CROP * CROP, C), lambda b: (b, 0))],
        out_specs=pl.BlockSpec((1, _RB, CROP, CROP, C),
                               lambda b: (0, b, 0, 0, 0)),
        out_shape=jax.ShapeDtypeStruct((1, TOP_K, CROP, CROP, C), jnp.float32),
    )(flat)


def _map_to_level(boxes):
    w = boxes[:, 2] - boxes[:, 0]
    h = boxes[:, 3] - boxes[:, 1]
    size = jnp.sqrt(w * h)
    levels = jnp.floor(1.0 + jnp.log2(size / 224.0 + EPS))
    return jnp.clip(levels, 0.0, 4.0)


def kernel(image_shape, boxes, classification, p0, p1, p2, p3, p4):
    table = jnp.concatenate(
        [p.reshape(-1, C) for p in (p0[0], p1[0], p2[0], p3[0], p4[0])], axis=0)

    b = boxes[0]
    cls = classification[0]
    scores = jnp.max(cls, axis=1)
    _, idx = lax.top_k(scores, TOP_K)
    b = jnp.take(b, idx, axis=0)
    cls = jnp.take(cls, idx, axis=0)
    levels = _map_to_level(b)
    order = jnp.argsort(levels, stable=True)
    b = jnp.take(b, order, axis=0)
    cls = jnp.take(cls, order, axis=0)
    levels = jnp.take(levels, order, axis=0)

    Hf = image_shape[1].astype(jnp.float32)
    Wf = image_shape[2].astype(jnp.float32)
    y1 = b[:, 1] / Hf
    x1 = b[:, 0] / Wf
    y2 = b[:, 3] / Hf
    x2 = b[:, 2] / Wf

    lev_i = levels.astype(jnp.int32)
    Hl = jnp.take(jnp.array(LEVEL_H, jnp.float32), lev_i)
    lbase = jnp.take(jnp.array(LEVEL_BASE, jnp.int32), lev_i)
    Wl_i = jnp.take(jnp.array(LEVEL_H, jnp.int32), lev_i)

    iy = jnp.arange(CROP, dtype=jnp.float32) / float(CROP - 1)
    ys = y1[:, None] * (Hl[:, None] - 1.0) + (y2 - y1)[:, None] * (Hl[:, None] - 1.0) * iy[None, :]
    xs = x1[:, None] * (Hl[:, None] - 1.0) + (x2 - x1)[:, None] * (Hl[:, None] - 1.0) * iy[None, :]
    y0f = jnp.floor(ys)
    x0f = jnp.floor(xs)
    Hi = Wl_i[:, None]
    y0 = jnp.clip(y0f.astype(jnp.int32), 0, Hi - 1)
    y1i = jnp.clip(y0 + 1, 0, Hi - 1)
    x0 = jnp.clip(x0f.astype(jnp.int32), 0, Hi - 1)
    x1i = jnp.clip(x0 + 1, 0, Hi - 1)
    wy = ys - y0f
    wx = xs - x0f

    row0 = lbase[:, None] + y0 * Wl_i[:, None]
    row1 = lbase[:, None] + y1i * Wl_i[:, None]
    i00 = row0[:, :, None] + x0[:, None, :]
    i01 = row0[:, :, None] + x1i[:, None, :]
    i10 = row1[:, :, None] + x0[:, None, :]
    i11 = row1[:, :, None] + x1i[:, None, :]
    wyc = wy[:, :, None]
    wxc = wx[:, None, :]
    w00 = (1.0 - wyc) * (1.0 - wxc)
    w01 = (1.0 - wyc) * wxc
    w10 = wyc * (1.0 - wxc)
    w11 = wyc * wxc

    def flat_i(a):
        return a.reshape(S).astype(jnp.int32)

    def flat_w(a):
        return jnp.broadcast_to(a, (TOP_K, CROP, CROP)).reshape(S)

    out = _roi_gather(table,
                      flat_i(i00), flat_i(i01), flat_i(i10), flat_i(i11),
                      flat_w(w00), flat_w(w01), flat_w(w10), flat_w(w11))
    rois = out.reshape(1, TOP_K, CROP, CROP, C)
    return (b[None], cls[None], rois)


# speedup vs baseline: 2.3262x; 1.5463x over previous
import functools

import jax
import jax.numpy as jnp
from jax import lax
from jax.experimental import pallas as pl
from jax.experimental.pallas import tpu as pltpu
from jax.experimental.pallas import tpu_sc as plsc

TOP_K = 500
CROP = 14
C = 256
EPS = 1e-7

LEVEL_H = (64, 32, 16, 8, 4)
LEVEL_BASE = (0, 4096, 5120, 5376, 5440)

NC = 2
NS = 16
NW = NC * NS

S = TOP_K * CROP * CROP
S_PER_W = 3072
CHUNK = 32
N_CHUNKS = S_PER_W // CHUNK


def _roi_body(table, i00, i01, i10, i11, w00, w01, w10, w11, out,
              ia0, ia1, ia2, ia3, wa0, wa1, wa2, wa3,
              va0, va1, va2, va3, vb0, vb1, vb2, vb3, ob, sem_a, sem_b):
    wid = lax.axis_index("s") * NC + lax.axis_index("c")
    start = jnp.minimum(wid * S_PER_W, S - S_PER_W)
    sla = pl.ds(start, S_PER_W)
    pltpu.sync_copy(i00.at[sla], ia0)
    pltpu.sync_copy(i01.at[sla], ia1)
    pltpu.sync_copy(i10.at[sla], ia2)
    pltpu.sync_copy(i11.at[sla], ia3)
    slw = pl.ds(0, S_PER_W)
    pltpu.sync_copy(w00.at[sla], wa0.at[slw])
    pltpu.sync_copy(w01.at[sla], wa1.at[slw])
    pltpu.sync_copy(w10.at[sla], wa2.at[slw])
    pltpu.sync_copy(w11.at[sla], wa3.at[slw])

    sets = ((va0, va1, va2, va3, sem_a), (vb0, vb1, vb2, vb3, sem_b))
    ias = (ia0, ia1, ia2, ia3)

    def issue(ci, si):
        bufs = sets[si]
        islc = pl.ds(ci * CHUNK, CHUNK)
        for k in range(4):
            pltpu.async_copy(table.at[ias[k].at[islc]], bufs[k], bufs[4])

    def drain(si):
        bufs = sets[si]
        for k in range(4):
            pltpu.make_async_copy(table.at[pl.ds(0, CHUNK)], bufs[k], bufs[4]).wait()

    def blend_store(ci, si):
        bufs = sets[si]
        woff = ci * CHUNK

        @plsc.parallel_loop(0, CHUNK, 1, unroll=2)
        def sample_body(s):
            a = jnp.full((16,), wa0[pl.ds(woff + s, 16)][0], dtype=jnp.float32)
            b = jnp.full((16,), wa1[pl.ds(woff + s, 16)][0], dtype=jnp.float32)
            c = jnp.full((16,), wa2[pl.ds(woff + s, 16)][0], dtype=jnp.float32)
            d = jnp.full((16,), wa3[pl.ds(woff + s, 16)][0], dtype=jnp.float32)
            for cc in range(C // 16):
                csl = pl.ds(cc * 16, 16)
                acc = a * bufs[0][s, csl] + b * bufs[1][s, csl]
                acc = acc + c * bufs[2][s, csl] + d * bufs[3][s, csl]
                ob[pl.ds(s * C + cc * 16, 16)] = acc

        pltpu.sync_copy(ob, out.at[pl.ds((start + woff) * C, CHUNK * C)])

    issue(0, 0)

    def pair_body(i, carry):
        ci = 2 * i
        issue(ci + 1, 1)
        drain(0)
        blend_store(ci, 0)

        @pl.when(ci + 2 < N_CHUNKS)
        def _():
            issue(ci + 2, 0)

        drain(1)
        blend_store(ci + 1, 1)
        return carry

    lax.fori_loop(0, N_CHUNKS // 2, pair_body, 0, unroll=False)


@jax.jit
def _roi_gather(table, i00, i01, i10, i11, w00, w01, w10, w11):
    mesh = plsc.VectorSubcoreMesh(core_axis_name="c", subcore_axis_name="s",
                                  num_cores=NC)
    return pl.kernel(
        _roi_body,
        out_type=jax.ShapeDtypeStruct((S * C,), jnp.float32),
        mesh=mesh,
        scratch_types=[
            pltpu.VMEM((S_PER_W,), jnp.int32),
            pltpu.VMEM((S_PER_W,), jnp.int32),
            pltpu.VMEM((S_PER_W,), jnp.int32),
            pltpu.VMEM((S_PER_W,), jnp.int32),
            pltpu.VMEM((S_PER_W + 16,), jnp.float32),
            pltpu.VMEM((S_PER_W + 16,), jnp.float32),
            pltpu.VMEM((S_PER_W + 16,), jnp.float32),
            pltpu.VMEM((S_PER_W + 16,), jnp.float32),
            pltpu.VMEM((CHUNK, C), jnp.float32),
            pltpu.VMEM((CHUNK, C), jnp.float32),
            pltpu.VMEM((CHUNK, C), jnp.float32),
            pltpu.VMEM((CHUNK, C), jnp.float32),
            pltpu.VMEM((CHUNK, C), jnp.float32),
            pltpu.VMEM((CHUNK, C), jnp.float32),
            pltpu.VMEM((CHUNK, C), jnp.float32),
            pltpu.VMEM((CHUNK, C), jnp.float32),
            pltpu.VMEM((CHUNK * C,), jnp.float32),
            pltpu.SemaphoreType.DMA,
            pltpu.SemaphoreType.DMA,
        ],
    )(table, i00, i01, i10, i11, w00, w01, w10, w11)


_RB = 4


def _retile_body(flat_ref, out_ref):
    out_ref[...] = flat_ref[...].reshape(1, _RB, CROP, CROP, C)


@jax.jit
def _retile(flat):
    return pl.pallas_call(
        _retile_body,
        grid=(TOP_K // _RB,),
        in_specs=[pl.BlockSpec((_RB * CROP * CROP, C), lambda b: (b, 0))],
        out_specs=pl.BlockSpec((1, _RB, CROP, CROP, C),
                               lambda b: (0, b, 0, 0, 0)),
        out_shape=jax.ShapeDtypeStruct((1, TOP_K, CROP, CROP, C), jnp.float32),
    )(flat)


def _map_to_level(boxes):
    w = boxes[:, 2] - boxes[:, 0]
    h = boxes[:, 3] - boxes[:, 1]
    size = jnp.sqrt(w * h)
    levels = jnp.floor(1.0 + jnp.log2(size / 224.0 + EPS))
    return jnp.clip(levels, 0.0, 4.0)


def kernel(image_shape, boxes, classification, p0, p1, p2, p3, p4):
    table = jnp.concatenate(
        [p.reshape(-1, C) for p in (p0[0], p1[0], p2[0], p3[0], p4[0])], axis=0)

    b = boxes[0]
    cls = classification[0]
    scores = jnp.max(cls, axis=1)
    _, idx = lax.top_k(scores, TOP_K)
    b = jnp.take(b, idx, axis=0)
    cls = jnp.take(cls, idx, axis=0)
    levels = _map_to_level(b)
    order = jnp.argsort(levels, stable=True)
    b = jnp.take(b, order, axis=0)
    cls = jnp.take(cls, order, axis=0)
    levels = jnp.take(levels, order, axis=0)

    Hf = image_shape[1].astype(jnp.float32)
    Wf = image_shape[2].astype(jnp.float32)
    y1 = b[:, 1] / Hf
    x1 = b[:, 0] / Wf
    y2 = b[:, 3] / Hf
    x2 = b[:, 2] / Wf

    lev_i = levels.astype(jnp.int32)
    Hl = jnp.take(jnp.array(LEVEL_H, jnp.float32), lev_i)
    lbase = jnp.take(jnp.array(LEVEL_BASE, jnp.int32), lev_i)
    Wl_i = jnp.take(jnp.array(LEVEL_H, jnp.int32), lev_i)

    iy = jnp.arange(CROP, dtype=jnp.float32) / float(CROP - 1)
    ys = y1[:, None] * (Hl[:, None] - 1.0) + (y2 - y1)[:, None] * (Hl[:, None] - 1.0) * iy[None, :]
    xs = x1[:, None] * (Hl[:, None] - 1.0) + (x2 - x1)[:, None] * (Hl[:, None] - 1.0) * iy[None, :]
    y0f = jnp.floor(ys)
    x0f = jnp.floor(xs)
    Hi = Wl_i[:, None]
    y0 = jnp.clip(y0f.astype(jnp.int32), 0, Hi - 1)
    y1i = jnp.clip(y0 + 1, 0, Hi - 1)
    x0 = jnp.clip(x0f.astype(jnp.int32), 0, Hi - 1)
    x1i = jnp.clip(x0 + 1, 0, Hi - 1)
    wy = ys - y0f
    wx = xs - x0f

    row0 = lbase[:, None] + y0 * Wl_i[:, None]
    row1 = lbase[:, None] + y1i * Wl_i[:, None]
    i00 = row0[:, :, None] + x0[:, None, :]
    i01 = row0[:, :, None] + x1i[:, None, :]
    i10 = row1[:, :, None] + x0[:, None, :]
    i11 = row1[:, :, None] + x1i[:, None, :]
    wyc = wy[:, :, None]
    wxc = wx[:, None, :]
    w00 = (1.0 - wyc) * (1.0 - wxc)
    w01 = (1.0 - wyc) * wxc
    w10 = wyc * (1.0 - wxc)
    w11 = wyc * wxc

    def flat_i(a):
        return a.reshape(S).astype(jnp.int32)

    def flat_w(a):
        return jnp.broadcast_to(a, (TOP_K, CROP, CROP)).reshape(S)

    out = _roi_gather(table,
                      flat_i(i00), flat_i(i01), flat_i(i10), flat_i(i11),
                      flat_w(w00), flat_w(w01), flat_w(w10), flat_w(w11))
    rois = out.reshape(1, TOP_K, CROP, CROP, C)
    return (b[None], cls[None], rois)


# speedup vs baseline: 2.7255x; 1.1717x over previous
import functools

import jax
import jax.numpy as jnp
from jax import lax
from jax.experimental import pallas as pl
from jax.experimental.pallas import tpu as pltpu
from jax.experimental.pallas import tpu_sc as plsc

TOP_K = 500
CROP = 14
C = 256
EPS = 1e-7

LEVEL_H = (64, 32, 16, 8, 4)
LEVEL_BASE = (0, 4096, 5120, 5376, 5440)

NC = 2
NS = 16
NW = NC * NS

S = TOP_K * CROP * CROP
S_PER_W = 3072
CHUNK = 64
N_CHUNKS = S_PER_W // CHUNK

_PERM = []
for _g in range(C // 32):
    for _k in range(16):
        _PERM.append(_g * 32 + _k)
        _PERM.append(_g * 32 + 16 + _k)


def _roi_body(table, i00, i01, i10, i11, w00, w01, w10, w11, out,
              ia0, ia1, ia2, ia3, wa0, wa1, wa2, wa3,
              va0, va1, va2, va3, vb0, vb1, vb2, vb3, ob, sem_a, sem_b):
    wid = lax.axis_index("s") * NC + lax.axis_index("c")
    start = jnp.minimum(wid * S_PER_W, S - S_PER_W)
    sla = pl.ds(start, S_PER_W)
    pltpu.sync_copy(i00.at[sla], ia0)
    pltpu.sync_copy(i01.at[sla], ia1)
    pltpu.sync_copy(i10.at[sla], ia2)
    pltpu.sync_copy(i11.at[sla], ia3)
    slw = pl.ds(0, S_PER_W)
    pltpu.sync_copy(w00.at[sla], wa0.at[slw])
    pltpu.sync_copy(w01.at[sla], wa1.at[slw])
    pltpu.sync_copy(w10.at[sla], wa2.at[slw])
    pltpu.sync_copy(w11.at[sla], wa3.at[slw])

    sets = ((va0, va1, va2, va3, sem_a), (vb0, vb1, vb2, vb3, sem_b))
    ias = (ia0, ia1, ia2, ia3)

    def issue(ci, si):
        bufs = sets[si]
        islc = pl.ds(ci * CHUNK, CHUNK)
        for k in range(4):
            pltpu.async_copy(table.at[ias[k].at[islc]], bufs[k], bufs[4])

    def drain(si):
        bufs = sets[si]
        for k in range(4):
            pltpu.make_async_copy(table.at[pl.ds(0, CHUNK)], bufs[k], bufs[4]).wait()

    def blend_store(ci, si):
        bufs = sets[si]
        woff = ci * CHUNK

        @plsc.parallel_loop(0, CHUNK, 1, unroll=2)
        def sample_body(s):
            a = jnp.full((16,), wa0[pl.ds(woff + s, 16)][0], dtype=jnp.float32)
            b = jnp.full((16,), wa1[pl.ds(woff + s, 16)][0], dtype=jnp.float32)
            c = jnp.full((16,), wa2[pl.ds(woff + s, 16)][0], dtype=jnp.float32)
            d = jnp.full((16,), wa3[pl.ds(woff + s, 16)][0], dtype=jnp.float32)
            for cc in range(C // 32):
                csl = pl.ds(cc * 16, 16)
                v0a, v0b = plsc.unpack(plsc.bitcast(bufs[0][s, csl], jnp.bfloat16), format=plsc.PackFormat.INTERLEAVED)
                v1a, v1b = plsc.unpack(plsc.bitcast(bufs[1][s, csl], jnp.bfloat16), format=plsc.PackFormat.INTERLEAVED)
                v2a, v2b = plsc.unpack(plsc.bitcast(bufs[2][s, csl], jnp.bfloat16), format=plsc.PackFormat.INTERLEAVED)
                v3a, v3b = plsc.unpack(plsc.bitcast(bufs[3][s, csl], jnp.bfloat16), format=plsc.PackFormat.INTERLEAVED)
                acca = (a * v0a + b * v1a) + (c * v2a + d * v3a)
                accb = (a * v0b + b * v1b) + (c * v2b + d * v3b)
                ob[pl.ds(s * C + cc * 32, 16)] = acca
                ob[pl.ds(s * C + cc * 32 + 16, 16)] = accb

        pltpu.sync_copy(ob, out.at[pl.ds((start + woff) * C, CHUNK * C)])

    issue(0, 0)

    def pair_body(i, carry):
        ci = 2 * i
        issue(ci + 1, 1)
        drain(0)
        blend_store(ci, 0)

        @pl.when(ci + 2 < N_CHUNKS)
        def _():
            issue(ci + 2, 0)

        drain(1)
        blend_store(ci + 1, 1)
        return carry

    lax.fori_loop(0, N_CHUNKS // 2, pair_body, 0, unroll=False)


@jax.jit
def _roi_gather(table, i00, i01, i10, i11, w00, w01, w10, w11):
    mesh = plsc.VectorSubcoreMesh(core_axis_name="c", subcore_axis_name="s",
                                  num_cores=NC)
    return pl.kernel(
        _roi_body,
        out_type=jax.ShapeDtypeStruct((S * C,), jnp.float32),
        mesh=mesh,
        compiler_params=pltpu.CompilerParams(needs_layout_passes=False),
        scratch_types=[
            pltpu.VMEM((S_PER_W,), jnp.int32),
            pltpu.VMEM((S_PER_W,), jnp.int32),
            pltpu.VMEM((S_PER_W,), jnp.int32),
            pltpu.VMEM((S_PER_W,), jnp.int32),
            pltpu.VMEM((S_PER_W + 16,), jnp.float32),
            pltpu.VMEM((S_PER_W + 16,), jnp.float32),
            pltpu.VMEM((S_PER_W + 16,), jnp.float32),
            pltpu.VMEM((S_PER_W + 16,), jnp.float32),
            pltpu.VMEM((CHUNK, C // 2), jnp.float32),
            pltpu.VMEM((CHUNK, C // 2), jnp.float32),
            pltpu.VMEM((CHUNK, C // 2), jnp.float32),
            pltpu.VMEM((CHUNK, C // 2), jnp.float32),
            pltpu.VMEM((CHUNK, C // 2), jnp.float32),
            pltpu.VMEM((CHUNK, C // 2), jnp.float32),
            pltpu.VMEM((CHUNK, C // 2), jnp.float32),
            pltpu.VMEM((CHUNK, C // 2), jnp.float32),
            pltpu.VMEM((CHUNK * C,), jnp.float32),
            pltpu.SemaphoreType.DMA,
            pltpu.SemaphoreType.DMA,
        ],
    )(table, i00, i01, i10, i11, w00, w01, w10, w11)


_RB = 4


def _retile_body(flat_ref, out_ref):
    out_ref[...] = flat_ref[...].reshape(1, _RB, CROP, CROP, C)


@jax.jit
def _retile(flat):
    return pl.pallas_call(
        _retile_body,
        grid=(TOP_K // _RB,),
        in_specs=[pl.BlockSpec((_RB * CROP * CROP, C), lambda b: (b, 0))],
        out_specs=pl.BlockSpec((1, _RB, CROP, CROP, C),
                               lambda b: (0, b, 0, 0, 0)),
        out_shape=jax.ShapeDtypeStruct((1, TOP_K, CROP, CROP, C), jnp.float32),
    )(flat)


def _map_to_level(boxes):
    w = boxes[:, 2] - boxes[:, 0]
    h = boxes[:, 3] - boxes[:, 1]
    size = jnp.sqrt(w * h)
    levels = jnp.floor(1.0 + jnp.log2(size / 224.0 + EPS))
    return jnp.clip(levels, 0.0, 4.0)


def kernel(image_shape, boxes, classification, p0, p1, p2, p3, p4):
    table = jnp.concatenate(
        [p.reshape(-1, C) for p in (p0[0], p1[0], p2[0], p3[0], p4[0])], axis=0)
    table = table[:, jnp.array(_PERM, jnp.int32)].astype(jnp.bfloat16)
    table = lax.bitcast_convert_type(table.reshape(-1, C // 2, 2), jnp.float32)

    b = boxes[0]
    cls = classification[0]
    scores = jnp.max(cls, axis=1)
    _, idx = lax.top_k(scores, TOP_K)
    b = jnp.take(b, idx, axis=0)
    cls = jnp.take(cls, idx, axis=0)
    levels = _map_to_level(b)
    order = jnp.argsort(levels, stable=True)
    b = jnp.take(b, order, axis=0)
    cls = jnp.take(cls, order, axis=0)
    levels = jnp.take(levels, order, axis=0)

    Hf = image_shape[1].astype(jnp.float32)
    Wf = image_shape[2].astype(jnp.float32)
    y1 = b[:, 1] / Hf
    x1 = b[:, 0] / Wf
    y2 = b[:, 3] / Hf
    x2 = b[:, 2] / Wf

    lev_i = levels.astype(jnp.int32)
    Hl = jnp.take(jnp.array(LEVEL_H, jnp.float32), lev_i)
    lbase = jnp.take(jnp.array(LEVEL_BASE, jnp.int32), lev_i)
    Wl_i = jnp.take(jnp.array(LEVEL_H, jnp.int32), lev_i)

    iy = jnp.arange(CROP, dtype=jnp.float32) / float(CROP - 1)
    ys = y1[:, None] * (Hl[:, None] - 1.0) + (y2 - y1)[:, None] * (Hl[:, None] - 1.0) * iy[None, :]
    xs = x1[:, None] * (Hl[:, None] - 1.0) + (x2 - x1)[:, None] * (Hl[:, None] - 1.0) * iy[None, :]
    y0f = jnp.floor(ys)
    x0f = jnp.floor(xs)
    Hi = Wl_i[:, None]
    y0 = jnp.clip(y0f.astype(jnp.int32), 0, Hi - 1)
    y1i = jnp.clip(y0 + 1, 0, Hi - 1)
    x0 = jnp.clip(x0f.astype(jnp.int32), 0, Hi - 1)
    x1i = jnp.clip(x0 + 1, 0, Hi - 1)
    wy = ys - y0f
    wx = xs - x0f

    row0 = lbase[:, None] + y0 * Wl_i[:, None]
    row1 = lbase[:, None] + y1i * Wl_i[:, None]
    i00 = row0[:, :, None] + x0[:, None, :]
    i01 = row0[:, :, None] + x1i[:, None, :]
    i10 = row1[:, :, None] + x0[:, None, :]
    i11 = row1[:, :, None] + x1i[:, None, :]
    wyc = wy[:, :, None]
    wxc = wx[:, None, :]
    w00 = (1.0 - wyc) * (1.0 - wxc)
    w01 = (1.0 - wyc) * wxc
    w10 = wyc * (1.0 - wxc)
    w11 = wyc * wxc

    def flat_i(a):
        return a.reshape(S).astype(jnp.int32)

    def flat_w(a):
        return jnp.broadcast_to(a, (TOP_K, CROP, CROP)).reshape(S)

    out = _roi_gather(table,
                      flat_i(i00), flat_i(i01), flat_i(i10), flat_i(i11),
                      flat_w(w00), flat_w(w01), flat_w(w10), flat_w(w11))
    rois = out.reshape(1, TOP_K, CROP, CROP, C)
    return (b[None], cls[None], rois)


# speedup vs baseline: 2.7600x; 1.0126x over previous
import functools

import jax
import jax.numpy as jnp
from jax import lax
from jax.experimental import pallas as pl
from jax.experimental.pallas import tpu as pltpu
from jax.experimental.pallas import tpu_sc as plsc

TOP_K = 500
CROP = 14
C = 256
EPS = 1e-7

LEVEL_H = (64, 32, 16, 8, 4)
LEVEL_BASE = (0, 4096, 5120, 5376, 5440)

NC = 2
NS = 16
NW = NC * NS

S = TOP_K * CROP * CROP
S_PER_W = 3072
CHUNK = 64
N_CHUNKS = S_PER_W // CHUNK

_PERM = []
for _g in range(C // 32):
    for _k in range(16):
        _PERM.append(_g * 32 + _k)
        _PERM.append(_g * 32 + 16 + _k)


def _roi_body(table, i00, i01, i10, i11, w00, w01, w10, w11, out,
              ia0, ia1, ia2, ia3, wa0, wa1, wa2, wa3,
              va0, va1, va2, va3, vb0, vb1, vb2, vb3, ob0, ob1,
              sem_a, sem_b, sem_o):
    wid = lax.axis_index("s") * NC + lax.axis_index("c")
    start = jnp.minimum(wid * S_PER_W, S - S_PER_W)
    sla = pl.ds(start, S_PER_W)
    pltpu.sync_copy(i00.at[sla], ia0)
    pltpu.sync_copy(i01.at[sla], ia1)
    pltpu.sync_copy(i10.at[sla], ia2)
    pltpu.sync_copy(i11.at[sla], ia3)
    slw = pl.ds(0, S_PER_W)
    pltpu.sync_copy(w00.at[sla], wa0.at[slw])
    pltpu.sync_copy(w01.at[sla], wa1.at[slw])
    pltpu.sync_copy(w10.at[sla], wa2.at[slw])
    pltpu.sync_copy(w11.at[sla], wa3.at[slw])

    sets = ((va0, va1, va2, va3, sem_a), (vb0, vb1, vb2, vb3, sem_b))
    obs = (ob0, ob1)
    ias = (ia0, ia1, ia2, ia3)

    def issue(ci, si):
        bufs = sets[si]
        islc = pl.ds(ci * CHUNK, CHUNK)
        for k in range(4):
            pltpu.async_copy(table.at[ias[k].at[islc]], bufs[k], bufs[4])

    def drain(si):
        bufs = sets[si]
        for k in range(4):
            pltpu.make_async_copy(table.at[pl.ds(0, CHUNK)], bufs[k], bufs[4]).wait()

    def blend_store(ci, si):
        bufs = sets[si]
        ob = obs[si]
        woff = ci * CHUNK

        @pl.when(ci >= 2)
        def _():
            pltpu.make_async_copy(out.at[pl.ds(0, CHUNK * C)], ob, sem_o).wait()

        @plsc.parallel_loop(0, CHUNK, 1, unroll=4)
        def sample_body(s):
            a = jnp.full((16,), wa0[pl.ds(woff + s, 16)][0], dtype=jnp.float32)
            b = jnp.full((16,), wa1[pl.ds(woff + s, 16)][0], dtype=jnp.float32)
            c = jnp.full((16,), wa2[pl.ds(woff + s, 16)][0], dtype=jnp.float32)
            d = jnp.full((16,), wa3[pl.ds(woff + s, 16)][0], dtype=jnp.float32)
            for cc in range(C // 32):
                csl = pl.ds(cc * 16, 16)
                v0a, v0b = plsc.unpack(plsc.bitcast(bufs[0][s, csl], jnp.bfloat16), format=plsc.PackFormat.INTERLEAVED)
                v1a, v1b = plsc.unpack(plsc.bitcast(bufs[1][s, csl], jnp.bfloat16), format=plsc.PackFormat.INTERLEAVED)
                v2a, v2b = plsc.unpack(plsc.bitcast(bufs[2][s, csl], jnp.bfloat16), format=plsc.PackFormat.INTERLEAVED)
                v3a, v3b = plsc.unpack(plsc.bitcast(bufs[3][s, csl], jnp.bfloat16), format=plsc.PackFormat.INTERLEAVED)
                acca = (a * v0a + b * v1a) + (c * v2a + d * v3a)
                accb = (a * v0b + b * v1b) + (c * v2b + d * v3b)
                ob[pl.ds(s * C + cc * 32, 16)] = acca
                ob[pl.ds(s * C + cc * 32 + 16, 16)] = accb

        pltpu.async_copy(ob, out.at[pl.ds((start + woff) * C, CHUNK * C)], sem_o)

    issue(0, 0)

    def pair_body(i, carry):
        ci = 2 * i
        issue(ci + 1, 1)
        drain(0)
        blend_store(ci, 0)

        @pl.when(ci + 2 < N_CHUNKS)
        def _():
            issue(ci + 2, 0)

        drain(1)
        blend_store(ci + 1, 1)
        return carry

    lax.fori_loop(0, N_CHUNKS // 2, pair_body, 0, unroll=False)
    pltpu.make_async_copy(out.at[pl.ds(0, CHUNK * C)], ob0, sem_o).wait()
    pltpu.make_async_copy(out.at[pl.ds(0, CHUNK * C)], ob1, sem_o).wait()


@jax.jit
def _roi_gather(table, i00, i01, i10, i11, w00, w01, w10, w11):
    mesh = plsc.VectorSubcoreMesh(core_axis_name="c", subcore_axis_name="s",
                                  num_cores=NC)
    return pl.kernel(
        _roi_body,
        out_type=jax.ShapeDtypeStruct((S * C,), jnp.float32),
        mesh=mesh,
        compiler_params=pltpu.CompilerParams(needs_layout_passes=False),
        scratch_types=[
            pltpu.VMEM((S_PER_W,), jnp.int32),
            pltpu.VMEM((S_PER_W,), jnp.int32),
            pltpu.VMEM((S_PER_W,), jnp.int32),
            pltpu.VMEM((S_PER_W,), jnp.int32),
            pltpu.VMEM((S_PER_W + 16,), jnp.float32),
            pltpu.VMEM((S_PER_W + 16,), jnp.float32),
            pltpu.VMEM((S_PER_W + 16,), jnp.float32),
            pltpu.VMEM((S_PER_W + 16,), jnp.float32),
            pltpu.VMEM((CHUNK, C // 2), jnp.float32),
            pltpu.VMEM((CHUNK, C // 2), jnp.float32),
            pltpu.VMEM((CHUNK, C // 2), jnp.float32),
            pltpu.VMEM((CHUNK, C // 2), jnp.float32),
            pltpu.VMEM((CHUNK, C // 2), jnp.float32),
            pltpu.VMEM((CHUNK, C // 2), jnp.float32),
            pltpu.VMEM((CHUNK, C // 2), jnp.float32),
            pltpu.VMEM((CHUNK, C // 2), jnp.float32),
            pltpu.VMEM((CHUNK * C,), jnp.float32),
            pltpu.VMEM((CHUNK * C,), jnp.float32),
            pltpu.SemaphoreType.DMA,
            pltpu.SemaphoreType.DMA,
            pltpu.SemaphoreType.DMA,
        ],
    )(table, i00, i01, i10, i11, w00, w01, w10, w11)


_RB = 4


def _retile_body(flat_ref, out_ref):
    out_ref[...] = flat_ref[...].reshape(1, _RB, CROP, CROP, C)


@jax.jit
def _retile(flat):
    return pl.pallas_call(
        _retile_body,
        grid=(TOP_K // _RB,),
        in_specs=[pl.BlockSpec((_RB * CROP * CROP, C), lambda b: (b, 0))],
        out_specs=pl.BlockSpec((1, _RB, CROP, CROP, C),
                               lambda b: (0, b, 0, 0, 0)),
        out_shape=jax.ShapeDtypeStruct((1, TOP_K, CROP, CROP, C), jnp.float32),
    )(flat)


def _map_to_level(boxes):
    w = boxes[:, 2] - boxes[:, 0]
    h = boxes[:, 3] - boxes[:, 1]
    size = jnp.sqrt(w * h)
    levels = jnp.floor(1.0 + jnp.log2(size / 224.0 + EPS))
    return jnp.clip(levels, 0.0, 4.0)


def kernel(image_shape, boxes, classification, p0, p1, p2, p3, p4):
    table = jnp.concatenate(
        [p.reshape(-1, C) for p in (p0[0], p1[0], p2[0], p3[0], p4[0])], axis=0)
    table = table[:, jnp.array(_PERM, jnp.int32)].astype(jnp.bfloat16)
    table = lax.bitcast_convert_type(table.reshape(-1, C // 2, 2), jnp.float32)

    b = boxes[0]
    cls = classification[0]
    scores = jnp.max(cls, axis=1)
    _, idx = lax.top_k(scores, TOP_K)
    b = jnp.take(b, idx, axis=0)
    cls = jnp.take(cls, idx, axis=0)
    levels = _map_to_level(b)
    order = jnp.argsort(levels, stable=True)
    b = jnp.take(b, order, axis=0)
    cls = jnp.take(cls, order, axis=0)
    levels = jnp.take(levels, order, axis=0)

    Hf = image_shape[1].astype(jnp.float32)
    Wf = image_shape[2].astype(jnp.float32)
    y1 = b[:, 1] / Hf
    x1 = b[:, 0] / Wf
    y2 = b[:, 3] / Hf
    x2 = b[:, 2] / Wf

    lev_i = levels.astype(jnp.int32)
    Hl = jnp.take(jnp.array(LEVEL_H, jnp.float32), lev_i)
    lbase = jnp.take(jnp.array(LEVEL_BASE, jnp.int32), lev_i)
    Wl_i = jnp.take(jnp.array(LEVEL_H, jnp.int32), lev_i)

    iy = jnp.arange(CROP, dtype=jnp.float32) / float(CROP - 1)
    ys = y1[:, None] * (Hl[:, None] - 1.0) + (y2 - y1)[:, None] * (Hl[:, None] - 1.0) * iy[None, :]
    xs = x1[:, None] * (Hl[:, None] - 1.0) + (x2 - x1)[:, None] * (Hl[:, None] - 1.0) * iy[None, :]
    y0f = jnp.floor(ys)
    x0f = jnp.floor(xs)
    Hi = Wl_i[:, None]
    y0 = jnp.clip(y0f.astype(jnp.int32), 0, Hi - 1)
    y1i = jnp.clip(y0 + 1, 0, Hi - 1)
    x0 = jnp.clip(x0f.astype(jnp.int32), 0, Hi - 1)
    x1i = jnp.clip(x0 + 1, 0, Hi - 1)
    wy = ys - y0f
    wx = xs - x0f

    row0 = lbase[:, None] + y0 * Wl_i[:, None]
    row1 = lbase[:, None] + y1i * Wl_i[:, None]
    i00 = row0[:, :, None] + x0[:, None, :]
    i01 = row0[:, :, None] + x1i[:, None, :]
    i10 = row1[:, :, None] + x0[:, None, :]
    i11 = row1[:, :, None] + x1i[:, None, :]
    wyc = wy[:, :, None]
    wxc = wx[:, None, :]
    w00 = (1.0 - wyc) * (1.0 - wxc)
    w01 = (1.0 - wyc) * wxc
    w10 = wyc * (1.0 - wxc)
    w11 = wyc * wxc

    def flat_i(a):
        return a.reshape(S).astype(jnp.int32)

    def flat_w(a):
        return jnp.broadcast_to(a, (TOP_K, CROP, CROP)).reshape(S)

    out = _roi_gather(table,
                      flat_i(i00), flat_i(i01), flat_i(i10), flat_i(i11),
                      flat_w(w00), flat_w(w01), flat_w(w10), flat_w(w11))
    rois = out.reshape(1, TOP_K, CROP, CROP, C)
    return (b[None], cls[None], rois)


# speedup vs baseline: 3.0289x; 1.0975x over previous
import functools

import jax
import jax.numpy as jnp
from jax import lax
from jax.experimental import pallas as pl
from jax.experimental.pallas import tpu as pltpu
from jax.experimental.pallas import tpu_sc as plsc

TOP_K = 500
CROP = 14
C = 256
EPS = 1e-7

LEVEL_H = (64, 32, 16, 8, 4)
LEVEL_BASE = (0, 4096, 5120, 5376, 5440)

NC = 2
NS = 16
NW = NC * NS

S = TOP_K * CROP * CROP
S_PER_W = 3072
CHUNK = 64
N_CHUNKS = S_PER_W // CHUNK

_PERM = []
for _g in range(C // 32):
    for _k in range(16):
        _PERM.append(_g * 32 + _k)
        _PERM.append(_g * 32 + 16 + _k)


def _roi_body(table, i00, dx, dr, wx, wy, out,
              ia0, da, ra, wxa, wya,
              ja1, ja2, ja3, jb1, jb2, jb3,
              va0, va1, va2, va3, vb0, vb1, vb2, vb3, ob0, ob1,
              sem_a, sem_b, sem_o):
    wid = lax.axis_index("s") * NC + lax.axis_index("c")
    start = jnp.minimum(wid * S_PER_W, S - S_PER_W)
    sla = pl.ds(start, S_PER_W)
    pltpu.sync_copy(i00.at[sla], ia0)
    pltpu.sync_copy(dx.at[sla], da)
    pltpu.sync_copy(dr.at[sla], ra)
    slw = pl.ds(0, S_PER_W)
    pltpu.sync_copy(wx.at[sla], wxa.at[slw])
    pltpu.sync_copy(wy.at[sla], wya.at[slw])

    sets = ((va0, va1, va2, va3, sem_a), (vb0, vb1, vb2, vb3, sem_b))
    obs = (ob0, ob1)
    jsets = ((ja1, ja2, ja3), (jb1, jb2, jb3))

    def issue(ci, si):
        bufs = sets[si]
        js = jsets[si]
        base = ci * CHUNK
        for k in range(CHUNK // 16):
            ksl = pl.ds(base + k * 16, 16)
            osl = pl.ds(k * 16, 16)
            v = ia0[ksl]
            d = da[ksl]
            r = ra[ksl]
            js[0][osl] = v + d
            js[1][osl] = v + r
            js[2][osl] = v + d + r
        islc = pl.ds(base, CHUNK)
        pltpu.async_copy(table.at[ia0.at[islc]], bufs[0], bufs[4])
        for k in range(3):
            pltpu.async_copy(table.at[js[k]], bufs[k + 1], bufs[4])

    def drain(si):
        bufs = sets[si]
        for k in range(4):
            pltpu.make_async_copy(table.at[pl.ds(0, CHUNK)], bufs[k], bufs[4]).wait()

    def blend_store(ci, si):
        bufs = sets[si]
        ob = obs[si]
        woff = ci * CHUNK

        @pl.when(ci >= 2)
        def _():
            pltpu.make_async_copy(out.at[pl.ds(0, CHUNK * C)], ob, sem_o).wait()

        @plsc.parallel_loop(0, CHUNK, 1, unroll=4)
        def sample_body(s):
            wxs = jnp.full((16,), wxa[pl.ds(woff + s, 16)][0], dtype=jnp.float32)
            wys = jnp.full((16,), wya[pl.ds(woff + s, 16)][0], dtype=jnp.float32)
            one = jnp.full((16,), 1.0, dtype=jnp.float32)
            a = (one - wys) * (one - wxs)
            b = (one - wys) * wxs
            c = wys * (one - wxs)
            d = wys * wxs
            for cc in range(C // 32):
                csl = pl.ds(cc * 16, 16)
                v0a, v0b = plsc.unpack(plsc.bitcast(bufs[0][s, csl], jnp.bfloat16), format=plsc.PackFormat.INTERLEAVED)
                v1a, v1b = plsc.unpack(plsc.bitcast(bufs[1][s, csl], jnp.bfloat16), format=plsc.PackFormat.INTERLEAVED)
                v2a, v2b = plsc.unpack(plsc.bitcast(bufs[2][s, csl], jnp.bfloat16), format=plsc.PackFormat.INTERLEAVED)
                v3a, v3b = plsc.unpack(plsc.bitcast(bufs[3][s, csl], jnp.bfloat16), format=plsc.PackFormat.INTERLEAVED)
                acca = (a * v0a + b * v1a) + (c * v2a + d * v3a)
                accb = (a * v0b + b * v1b) + (c * v2b + d * v3b)
                ob[pl.ds(s * C + cc * 32, 16)] = acca
                ob[pl.ds(s * C + cc * 32 + 16, 16)] = accb

        pltpu.async_copy(ob, out.at[pl.ds((start + woff) * C, CHUNK * C)], sem_o)

    issue(0, 0)

    def pair_body(i, carry):
        ci = 2 * i
        issue(ci + 1, 1)
        drain(0)
        blend_store(ci, 0)

        @pl.when(ci + 2 < N_CHUNKS)
        def _():
            issue(ci + 2, 0)

        drain(1)
        blend_store(ci + 1, 1)
        return carry

    lax.fori_loop(0, N_CHUNKS // 2, pair_body, 0, unroll=False)
    pltpu.make_async_copy(out.at[pl.ds(0, CHUNK * C)], ob0, sem_o).wait()
    pltpu.make_async_copy(out.at[pl.ds(0, CHUNK * C)], ob1, sem_o).wait()


@jax.jit
def _roi_gather(table, i00, dx, dr, wx, wy):
    mesh = plsc.VectorSubcoreMesh(core_axis_name="c", subcore_axis_name="s",
                                  num_cores=NC)
    return pl.kernel(
        _roi_body,
        out_type=jax.ShapeDtypeStruct((S * C,), jnp.float32),
        mesh=mesh,
        compiler_params=pltpu.CompilerParams(needs_layout_passes=False),
        scratch_types=[
            pltpu.VMEM((S_PER_W,), jnp.int32),
            pltpu.VMEM((S_PER_W,), jnp.int32),
            pltpu.VMEM((S_PER_W,), jnp.int32),
            pltpu.VMEM((S_PER_W + 16,), jnp.float32),
            pltpu.VMEM((S_PER_W + 16,), jnp.float32),
            pltpu.VMEM((CHUNK,), jnp.int32),
            pltpu.VMEM((CHUNK,), jnp.int32),
            pltpu.VMEM((CHUNK,), jnp.int32),
            pltpu.VMEM((CHUNK,), jnp.int32),
            pltpu.VMEM((CHUNK,), jnp.int32),
            pltpu.VMEM((CHUNK,), jnp.int32),
            pltpu.VMEM((CHUNK, C // 2), jnp.float32),
            pltpu.VMEM((CHUNK, C // 2), jnp.float32),
            pltpu.VMEM((CHUNK, C // 2), jnp.float32),
            pltpu.VMEM((CHUNK, C // 2), jnp.float32),
            pltpu.VMEM((CHUNK, C // 2), jnp.float32),
            pltpu.VMEM((CHUNK, C // 2), jnp.float32),
            pltpu.VMEM((CHUNK, C // 2), jnp.float32),
            pltpu.VMEM((CHUNK, C // 2), jnp.float32),
            pltpu.VMEM((CHUNK * C,), jnp.float32),
            pltpu.VMEM((CHUNK * C,), jnp.float32),
            pltpu.SemaphoreType.DMA,
            pltpu.SemaphoreType.DMA,
            pltpu.SemaphoreType.DMA,
        ],
    )(table, i00, dx, dr, wx, wy)


_RB = 4


def _retile_body(flat_ref, out_ref):
    out_ref[...] = flat_ref[...].reshape(1, _RB, CROP, CROP, C)


@jax.jit
def _retile(flat):
    return pl.pallas_call(
        _retile_body,
        grid=(TOP_K // _RB,),
        in_specs=[pl.BlockSpec((_RB * CROP * CROP, C), lambda b: (b, 0))],
        out_specs=pl.BlockSpec((1, _RB, CROP, CROP, C),
                               lambda b: (0, b, 0, 0, 0)),
        out_shape=jax.ShapeDtypeStruct((1, TOP_K, CROP, CROP, C), jnp.float32),
    )(flat)


def _map_to_level(boxes):
    w = boxes[:, 2] - boxes[:, 0]
    h = boxes[:, 3] - boxes[:, 1]
    size = jnp.sqrt(w * h)
    levels = jnp.floor(1.0 + jnp.log2(size / 224.0 + EPS))
    return jnp.clip(levels, 0.0, 4.0)


def kernel(image_shape, boxes, classification, p0, p1, p2, p3, p4):
    table = jnp.concatenate(
        [p.reshape(-1, C) for p in (p0[0], p1[0], p2[0], p3[0], p4[0])], axis=0)
    table = table[:, jnp.array(_PERM, jnp.int32)].astype(jnp.bfloat16)
    table = lax.bitcast_convert_type(table.reshape(-1, C // 2, 2), jnp.float32)

    b = boxes[0]
    cls = classification[0]
    scores = jnp.max(cls, axis=1)
    _, idx = lax.top_k(scores, TOP_K)
    b = jnp.take(b, idx, axis=0)
    cls = jnp.take(cls, idx, axis=0)
    levels = _map_to_level(b)
    order = jnp.argsort(levels, stable=True)
    b = jnp.take(b, order, axis=0)
    cls = jnp.take(cls, order, axis=0)
    levels = jnp.take(levels, order, axis=0)

    Hf = image_shape[1].astype(jnp.float32)
    Wf = image_shape[2].astype(jnp.float32)
    y1 = b[:, 1] / Hf
    x1 = b[:, 0] / Wf
    y2 = b[:, 3] / Hf
    x2 = b[:, 2] / Wf

    lev_i = levels.astype(jnp.int32)
    Hl = jnp.take(jnp.array(LEVEL_H, jnp.float32), lev_i)
    lbase = jnp.take(jnp.array(LEVEL_BASE, jnp.int32), lev_i)
    Wl_i = jnp.take(jnp.array(LEVEL_H, jnp.int32), lev_i)

    iy = jnp.arange(CROP, dtype=jnp.float32) / float(CROP - 1)
    ys = y1[:, None] * (Hl[:, None] - 1.0) + (y2 - y1)[:, None] * (Hl[:, None] - 1.0) * iy[None, :]
    xs = x1[:, None] * (Hl[:, None] - 1.0) + (x2 - x1)[:, None] * (Hl[:, None] - 1.0) * iy[None, :]
    y0f = jnp.floor(ys)
    x0f = jnp.floor(xs)
    Hi = Wl_i[:, None]
    y0 = jnp.clip(y0f.astype(jnp.int32), 0, Hi - 1)
    y1i = jnp.clip(y0 + 1, 0, Hi - 1)
    x0 = jnp.clip(x0f.astype(jnp.int32), 0, Hi - 1)
    x1i = jnp.clip(x0 + 1, 0, Hi - 1)
    wy = ys - y0f
    wx = xs - x0f

    row0 = lbase[:, None] + y0 * Wl_i[:, None]
    row1 = lbase[:, None] + y1i * Wl_i[:, None]
    i00 = row0[:, :, None] + x0[:, None, :]
    wyc = wy[:, :, None]
    wxc = wx[:, None, :]

    def flat_i(a):
        return jnp.broadcast_to(a, (TOP_K, CROP, CROP)).reshape(S).astype(jnp.int32)

    def flat_w(a):
        return jnp.broadcast_to(a, (TOP_K, CROP, CROP)).reshape(S)

    dxv = (x1i - x0)[:, None, :]
    drv = (row1 - row0)[:, :, None]
    out = _roi_gather(table,
                      flat_i(i00), flat_i(dxv), flat_i(drv),
                      flat_w(wxc), flat_w(wyc))
    rois = out.reshape(1, TOP_K, CROP, CROP, C)
    return (b[None], cls[None], rois)


# speedup vs baseline: 3.0313x; 1.0008x over previous
import jax
import jax.numpy as jnp
from jax import lax
from jax.experimental import pallas as pl
from jax.experimental.pallas import tpu as pltpu
from jax.experimental.pallas import tpu_sc as plsc

TOP_K = 500
CROP = 14
C = 256
EPS = 1e-7

LEVEL_H = (64, 32, 16, 8, 4)
LEVEL_BASE = (0, 4096, 5120, 5376, 5440)

NC = 2
NS = 16
NW = NC * NS

S = TOP_K * CROP * CROP
S_PER_W = 3072
CHUNK = 64
N_CHUNKS = S_PER_W // CHUNK

_PERM = []
for _g in range(C // 32):
    for _k in range(16):
        _PERM.append(_g * 32 + _k)
        _PERM.append(_g * 32 + 16 + _k)


def _roi_body(table, i00, dx, dr, wx, wy, out,
              ia0, da, ra, wxa, wya,
              ja1, ja2, ja3, jb1, jb2, jb3,
              va0, va1, va2, va3, vb0, vb1, vb2, vb3, ob0, ob1,
              sem_a, sem_b, sem_o):
    wid = lax.axis_index("s") * NC + lax.axis_index("c")
    start = jnp.minimum(wid * S_PER_W, S - S_PER_W)
    sla = pl.ds(start, S_PER_W)
    pltpu.sync_copy(i00.at[sla], ia0)
    pltpu.sync_copy(dx.at[sla], da)
    pltpu.sync_copy(dr.at[sla], ra)
    slw = pl.ds(0, S_PER_W)
    pltpu.sync_copy(wx.at[sla], wxa.at[slw])
    pltpu.sync_copy(wy.at[sla], wya.at[slw])

    sets = ((va0, va1, va2, va3, sem_a), (vb0, vb1, vb2, vb3, sem_b))
    obs = (ob0, ob1)
    jsets = ((ja1, ja2, ja3), (jb1, jb2, jb3))

    def issue(ci, si):
        bufs = sets[si]
        js = jsets[si]
        base = ci * CHUNK
        for k in range(CHUNK // 16):
            ksl = pl.ds(base + k * 16, 16)
            osl = pl.ds(k * 16, 16)
            v = ia0[ksl]
            d = da[ksl]
            r = ra[ksl]
            js[0][osl] = v + d
            js[1][osl] = v + r
            js[2][osl] = v + d + r
        islc = pl.ds(base, CHUNK)
        pltpu.async_copy(table.at[ia0.at[islc]], bufs[0], bufs[4])
        for k in range(3):
            pltpu.async_copy(table.at[js[k]], bufs[k + 1], bufs[4])

    def drain(si):
        bufs = sets[si]
        for k in range(4):
            pltpu.make_async_copy(table.at[pl.ds(0, CHUNK)], bufs[k], bufs[4]).wait()

    def blend_store(ci, si):
        bufs = sets[si]
        ob = obs[si]
        woff = ci * CHUNK

        @pl.when(ci >= 2)
        def _():
            pltpu.make_async_copy(out.at[pl.ds(0, CHUNK * C)], ob, sem_o).wait()

        @plsc.parallel_loop(0, CHUNK, 1, unroll=4)
        def sample_body(s):
            wxs = jnp.full((16,), wxa[pl.ds(woff + s, 16)][0], dtype=jnp.float32)
            wys = jnp.full((16,), wya[pl.ds(woff + s, 16)][0], dtype=jnp.float32)
            one = jnp.full((16,), 1.0, dtype=jnp.float32)
            a = (one - wys) * (one - wxs)
            b = (one - wys) * wxs
            c = wys * (one - wxs)
            d = wys * wxs
            for cc in range(C // 32):
                csl = pl.ds(cc * 16, 16)
                v0a, v0b = plsc.unpack(plsc.bitcast(bufs[0][s, csl], jnp.bfloat16), format=plsc.PackFormat.INTERLEAVED)
                v1a, v1b = plsc.unpack(plsc.bitcast(bufs[1][s, csl], jnp.bfloat16), format=plsc.PackFormat.INTERLEAVED)
                v2a, v2b = plsc.unpack(plsc.bitcast(bufs[2][s, csl], jnp.bfloat16), format=plsc.PackFormat.INTERLEAVED)
                v3a, v3b = plsc.unpack(plsc.bitcast(bufs[3][s, csl], jnp.bfloat16), format=plsc.PackFormat.INTERLEAVED)
                acca = (a * v0a + b * v1a) + (c * v2a + d * v3a)
                accb = (a * v0b + b * v1b) + (c * v2b + d * v3b)
                ob[pl.ds(s * C + cc * 32, 16)] = acca
                ob[pl.ds(s * C + cc * 32 + 16, 16)] = accb

        pltpu.async_copy(ob, out.at[pl.ds((start + woff) * C, CHUNK * C)], sem_o)

    issue(0, 0)

    def pair_body(i, carry):
        ci = 2 * i
        issue(ci + 1, 1)
        drain(0)
        blend_store(ci, 0)

        @pl.when(ci + 2 < N_CHUNKS)
        def _():
            issue(ci + 2, 0)

        drain(1)
        blend_store(ci + 1, 1)
        return carry

    lax.fori_loop(0, N_CHUNKS // 2, pair_body, 0, unroll=False)
    pltpu.make_async_copy(out.at[pl.ds(0, CHUNK * C)], ob0, sem_o).wait()
    pltpu.make_async_copy(out.at[pl.ds(0, CHUNK * C)], ob1, sem_o).wait()


@jax.jit
def _roi_gather(table, i00, dx, dr, wx, wy):
    mesh = plsc.VectorSubcoreMesh(core_axis_name="c", subcore_axis_name="s",
                                  num_cores=NC)
    return pl.kernel(
        _roi_body,
        out_type=jax.ShapeDtypeStruct((S * C,), jnp.float32),
        mesh=mesh,
        compiler_params=pltpu.CompilerParams(needs_layout_passes=False),
        scratch_types=[
            pltpu.VMEM((S_PER_W,), jnp.int32),
            pltpu.VMEM((S_PER_W,), jnp.int32),
            pltpu.VMEM((S_PER_W,), jnp.int32),
            pltpu.VMEM((S_PER_W + 16,), jnp.float32),
            pltpu.VMEM((S_PER_W + 16,), jnp.float32),
            pltpu.VMEM((CHUNK,), jnp.int32),
            pltpu.VMEM((CHUNK,), jnp.int32),
            pltpu.VMEM((CHUNK,), jnp.int32),
            pltpu.VMEM((CHUNK,), jnp.int32),
            pltpu.VMEM((CHUNK,), jnp.int32),
            pltpu.VMEM((CHUNK,), jnp.int32),
            pltpu.VMEM((CHUNK, C // 2), jnp.float32),
            pltpu.VMEM((CHUNK, C // 2), jnp.float32),
            pltpu.VMEM((CHUNK, C // 2), jnp.float32),
            pltpu.VMEM((CHUNK, C // 2), jnp.float32),
            pltpu.VMEM((CHUNK, C // 2), jnp.float32),
            pltpu.VMEM((CHUNK, C // 2), jnp.float32),
            pltpu.VMEM((CHUNK, C // 2), jnp.float32),
            pltpu.VMEM((CHUNK, C // 2), jnp.float32),
            pltpu.VMEM((CHUNK * C,), jnp.float32),
            pltpu.VMEM((CHUNK * C,), jnp.float32),
            pltpu.SemaphoreType.DMA,
            pltpu.SemaphoreType.DMA,
            pltpu.SemaphoreType.DMA,
        ],
    )(table, i00, dx, dr, wx, wy)


def _map_to_level(boxes):
    w = boxes[:, 2] - boxes[:, 0]
    h = boxes[:, 3] - boxes[:, 1]
    size = jnp.sqrt(w * h)
    levels = jnp.floor(1.0 + jnp.log2(size / 224.0 + EPS))
    return jnp.clip(levels, 0.0, 4.0)


def kernel(image_shape, boxes, classification, p0, p1, p2, p3, p4):
    table = jnp.concatenate(
        [p.reshape(-1, C) for p in (p0[0], p1[0], p2[0], p3[0], p4[0])], axis=0)
    table = table[:, jnp.array(_PERM, jnp.int32)].astype(jnp.bfloat16)
    table = lax.bitcast_convert_type(table.reshape(-1, C // 2, 2), jnp.float32)

    b = boxes[0]
    cls = classification[0]
    scores = jnp.max(cls, axis=1)
    _, idx = lax.top_k(scores, TOP_K)
    b = jnp.take(b, idx, axis=0)
    cls = jnp.take(cls, idx, axis=0)
    levels = _map_to_level(b)
    order = jnp.argsort(levels, stable=True)
    b = jnp.take(b, order, axis=0)
    cls = jnp.take(cls, order, axis=0)
    levels = jnp.take(levels, order, axis=0)

    Hf = image_shape[1].astype(jnp.float32)
    Wf = image_shape[2].astype(jnp.float32)
    y1 = b[:, 1] / Hf
    x1 = b[:, 0] / Wf
    y2 = b[:, 3] / Hf
    x2 = b[:, 2] / Wf

    lev_i = levels.astype(jnp.int32)
    Hl = jnp.take(jnp.array(LEVEL_H, jnp.float32), lev_i)
    lbase = jnp.take(jnp.array(LEVEL_BASE, jnp.int32), lev_i)
    Wl_i = jnp.take(jnp.array(LEVEL_H, jnp.int32), lev_i)

    iy = jnp.arange(CROP, dtype=jnp.float32) / float(CROP - 1)
    ys = y1[:, None] * (Hl[:, None] - 1.0) + (y2 - y1)[:, None] * (Hl[:, None] - 1.0) * iy[None, :]
    xs = x1[:, None] * (Hl[:, None] - 1.0) + (x2 - x1)[:, None] * (Hl[:, None] - 1.0) * iy[None, :]
    y0f = jnp.floor(ys)
    x0f = jnp.floor(xs)
    Hi = Wl_i[:, None]
    y0 = jnp.clip(y0f.astype(jnp.int32), 0, Hi - 1)
    y1i = jnp.clip(y0 + 1, 0, Hi - 1)
    x0 = jnp.clip(x0f.astype(jnp.int32), 0, Hi - 1)
    x1i = jnp.clip(x0 + 1, 0, Hi - 1)
    wy = ys - y0f
    wx = xs - x0f

    row0 = lbase[:, None] + y0 * Wl_i[:, None]
    row1 = lbase[:, None] + y1i * Wl_i[:, None]
    i00 = row0[:, :, None] + x0[:, None, :]
    wyc = wy[:, :, None]
    wxc = wx[:, None, :]

    def flat_i(a):
        return jnp.broadcast_to(a, (TOP_K, CROP, CROP)).reshape(S).astype(jnp.int32)

    def flat_w(a):
        return jnp.broadcast_to(a, (TOP_K, CROP, CROP)).reshape(S)

    dxv = (x1i - x0)[:, None, :]
    drv = (row1 - row0)[:, :, None]
    out = _roi_gather(table,
                      flat_i(i00), flat_i(dxv), flat_i(drv),
                      flat_w(wxc), flat_w(wyc))
    rois = out.reshape(1, TOP_K, CROP, CROP, C)
    return (b[None], cls[None], rois)
